# Initial kernel scaffold; baseline (speedup 1.0000x reference)
import jax
import jax.numpy as jnp
from jax.experimental import pallas as pl


def kernel(spike_input):
    raise NotImplementedError("write your pallas kernel here")


# speedup vs baseline: 100.8572x; 100.8572x over previous
import functools

import jax
import jax.numpy as jnp
from jax import lax
from jax.experimental import pallas as pl
from jax.experimental.pallas import tpu as pltpu
from jax.experimental.pallas import tpu_sc as plsc

_T = 2048
_B = 64
_N = 256
_C = _B * _N
_K = 16
_DT = 0.001

_NC = 2
_NS = 16
_NW = _NC * _NS
_CPW = _C // _NW
_G = _CPW // 16
_TC0 = 64
_SB = 16


def _make_decoder():
    mesh = plsc.VectorSubcoreMesh(core_axis_name="c", subcore_axis_name="s")

    @functools.partial(
        pl.kernel,
        out_type=jax.ShapeDtypeStruct((_K, _C), jnp.float32),
        mesh=mesh,
        scratch_types=[
            pltpu.VMEM((_TC0, _CPW), jnp.float32),
            pltpu.VMEM((_K, _CPW), jnp.float32),
            pltpu.VMEM((_TC0, 128), jnp.float32),
        ],
        compiler_params=pltpu.CompilerParams(needs_layout_passes=False),
    )
    def decode(x_hbm, out_hbm, chunk_v, out_v, rare_v):
        wid = lax.axis_index("s") * _NC + lax.axis_index("c")
        cbase = pl.multiple_of(wid * _CPW, _CPW)

        pltpu.sync_copy(x_hbm.at[pl.ds(0, _TC0), pl.ds(cbase, _CPW)], chunk_v)

        lane = jnp.arange(16, dtype=jnp.int32)
        inf_v = jnp.full((16,), jnp.inf, dtype=jnp.float32)

        def init_body(j, carry):
            for k in range(_K):
                out_v[k, pl.ds(j * 16, 16)] = inf_v
            return carry

        lax.fori_loop(0, _G, init_body, 0)

        def scan_step(src_ref, row, col16, chl, t_scalar, cnt):
            v = src_ref[row, pl.ds(col16, 16)]
            m = (v > 0.0) & (cnt < _K)
            slot = jnp.minimum(cnt, _K - 1)
            val = jnp.broadcast_to(t_scalar.astype(jnp.float32) * _DT, (16,))
            plsc.store_scatter(out_v, [slot, chl], val, mask=m)
            return cnt + m.astype(jnp.int32)

        def group_body(g, carry):
            col = g * 16
            chl = lane + col

            def pre_cond(state):
                t, cnt = state
                return (t < _TC0) & (jnp.min(cnt) < _K)

            def pre_body(state):
                t, cnt = state
                for d in range(_SB):
                    cnt = scan_step(chunk_v, t + d, col, chl, t + d, cnt)
                return t + _SB, cnt

            t, cnt = lax.while_loop(
                pre_cond, pre_body,
                (jnp.int32(0), jnp.zeros((16,), jnp.int32)))

            def rare_cond(state):
                t, cnt = state
                return (t < _T) & (jnp.min(cnt) < _K)

            col128 = pl.multiple_of((col // 128) * 128, 128)
            sub = col - col128

            def rare_body(state):
                t, cnt = state
                pltpu.sync_copy(
                    x_hbm.at[pl.ds(pl.multiple_of(t, _TC0), _TC0),
                             pl.ds(cbase + col128, 128)],
                    rare_v)

                def inner(i, cnt):
                    return scan_step(rare_v, i, sub, chl, t + i, cnt)

                cnt = lax.fori_loop(0, _TC0, inner, cnt)
                return t + _TC0, cnt

            lax.while_loop(rare_cond, rare_body, (t, cnt))
            return carry

        lax.fori_loop(0, _G, group_body, 0)

        pltpu.sync_copy(out_v, out_hbm.at[:, pl.ds(cbase, _CPW)])

    return decode


_decoder = _make_decoder()


def kernel(spike_input):
    x = spike_input.reshape(_T, _C)
    out = _decoder(x)
    return out.reshape(_K, _B, _N)


# speedup vs baseline: 321.6846x; 3.1895x over previous
import functools

import jax
import jax.numpy as jnp
from jax import lax
from jax.experimental import pallas as pl
from jax.experimental.pallas import tpu as pltpu
from jax.experimental.pallas import tpu_sc as plsc

_T = 2048
_B = 64
_N = 256
_K = 16
_DT = 0.001

_TP = 96
_SB = 16
_TB = 32
_NG = 32


def _make_decoder():
    mesh = plsc.VectorSubcoreMesh(core_axis_name="c", subcore_axis_name="s")

    @functools.partial(
        pl.kernel,
        out_type=jax.ShapeDtypeStruct((_B, _K, _N), jnp.float32),
        mesh=mesh,
        scratch_types=[
            pltpu.VMEM((_TP, 8, 128), jnp.float32),
            pltpu.VMEM((4, _K, 128), jnp.float32),
            pltpu.VMEM((_NG, 16), jnp.int32),
        ],
        compiler_params=pltpu.CompilerParams(needs_layout_passes=False),
    )
    def decode(x_hbm, out_hbm, chunk_v, out_v, cnt_v):
        core = lax.axis_index("c")
        sub = lax.axis_index("s")
        b0 = pl.multiple_of((sub % 8) * 8, 8)
        n0 = pl.multiple_of((sub // 8) * 128, 128)
        bh = core * 4

        pltpu.sync_copy(
            x_hbm.at[pl.ds(0, _TP), pl.ds(b0, 8), pl.ds(n0, 128)], chunk_v)

        lane = jnp.arange(16, dtype=jnp.int32)
        inf_v = jnp.full((16,), jnp.inf, dtype=jnp.float32)

        def scan_step(row, b_loc, n_off, b_rel_v, n_idx, t_scalar, cnt):
            v = chunk_v[row, b_loc, pl.ds(n_off, 16)]
            m = (v > 0.0) & (cnt < _K)
            slot = jnp.minimum(cnt, _K - 1)
            val = jnp.broadcast_to(t_scalar.astype(jnp.float32) * _DT, (16,))
            plsc.store_scatter(out_v, [b_rel_v, slot, n_idx], val, mask=m)
            return cnt + m.astype(jnp.int32)

        def group_geom(g):
            b_rel = g // 8
            n_off = (g % 8) * 16
            b_loc = bh + b_rel
            b_rel_v = jnp.broadcast_to(b_rel, (16,)).astype(jnp.int32)
            n_idx = n_off + lane
            return b_loc, n_off, b_rel_v, n_idx

        def group_body(g, mask):
            b_loc, n_off, b_rel_v, n_idx = group_geom(g)

            def pre_cond(state):
                t, cnt = state
                return (t < _TP) & (jnp.min(cnt) < _K)

            def pre_body(state):
                t, cnt = state
                for d in range(_SB):
                    cnt = scan_step(t + d, b_loc, n_off, b_rel_v, n_idx,
                                    t + d, cnt)
                return t + _SB, cnt

            _, cnt = lax.while_loop(
                pre_cond, pre_body,
                (jnp.int32(0), jnp.zeros((16,), jnp.int32)))
            cnt_v[g, :] = cnt
            short = (jnp.min(cnt) < _K).astype(jnp.int32)
            return mask | (short << g)

        mask = lax.fori_loop(0, _NG, group_body, jnp.int32(0))

        def rare_cond(state):
            t, mask = state
            return (t < _T) & (mask != 0)

        def rare_body(state):
            t, mask = state
            pltpu.sync_copy(
                x_hbm.at[pl.ds(t, _TB), pl.ds(b0, 8), pl.ds(n0, 128)],
                chunk_v.at[pl.ds(0, _TB)])

            def gb(g, mk):
                def live(mk):
                    b_loc, n_off, b_rel_v, n_idx = group_geom(g)

                    def inner(i, cnt):
                        return scan_step(i, b_loc, n_off, b_rel_v, n_idx,
                                         t + i, cnt)

                    cnt = lax.fori_loop(0, _TB, inner, cnt_v[g, :])
                    cnt_v[g, :] = cnt
                    done = jnp.min(cnt) >= _K
                    return mk & ~jnp.where(done, jnp.int32(1) << g,
                                           jnp.int32(0))

                return lax.cond((mk >> g) & 1 != 0, live, lambda m: m, mk)

            mask = lax.fori_loop(0, _NG, gb, mask)
            return t + _TB, mask

        _, mask = lax.while_loop(rare_cond, rare_body, (jnp.int32(_TP), mask))

        def fill_body(g, mk):
            def live(mk):
                _, _, b_rel_v, n_idx = group_geom(g)
                cnt = cnt_v[g, :]
                for slot in range(_K):
                    m = cnt <= slot
                    slot_v = jnp.broadcast_to(slot, (16,)).astype(jnp.int32)
                    plsc.store_scatter(out_v, [b_rel_v, slot_v, n_idx],
                                       inf_v, mask=m)
                return mk

            return lax.cond((mk >> g) & 1 != 0, live, lambda m: m, mk)

        lax.fori_loop(0, _NG, fill_body, mask)

        pltpu.sync_copy(
            out_v, out_hbm.at[pl.ds(b0 + bh, 4), :, pl.ds(n0, 128)])

    return decode


_decoder = _make_decoder()


def kernel(spike_input):
    out = _decoder(spike_input)
    return jnp.transpose(out, (1, 0, 2))


# speedup vs baseline: 439.7398x; 1.3670x over previous
import functools

import jax
import jax.numpy as jnp
from jax import lax
from jax.experimental import pallas as pl
from jax.experimental.pallas import tpu as pltpu
from jax.experimental.pallas import tpu_sc as plsc

_T = 2048
_B = 64
_N = 256
_K = 16
_DT = 0.001

_TP = 96
_SB = 16
_TB = 32
_NG = 32
_KP = 32


def _make_decoder():
    mesh = plsc.VectorSubcoreMesh(core_axis_name="c", subcore_axis_name="s")

    @functools.partial(
        pl.kernel,
        out_type=jax.ShapeDtypeStruct((_B, _KP, _N), jnp.float32),
        mesh=mesh,
        scratch_types=[
            pltpu.VMEM((_TP, 8, 128), jnp.float32),
            pltpu.VMEM((4, _KP, 128), jnp.float32),
            pltpu.VMEM((_NG, 16), jnp.int32),
        ],
        compiler_params=pltpu.CompilerParams(needs_layout_passes=False),
    )
    def decode(x_hbm, out_hbm, chunk_v, out_v, cnt_v):
        core = lax.axis_index("c")
        sub = lax.axis_index("s")
        b0 = pl.multiple_of((sub % 8) * 8, 8)
        n0 = pl.multiple_of((sub // 8) * 128, 128)
        bh = core * 4

        pltpu.sync_copy(
            x_hbm.at[pl.ds(0, _TP), pl.ds(b0, 8), pl.ds(n0, 128)], chunk_v)

        lane = jnp.arange(16, dtype=jnp.int32)
        inf_v = jnp.full((16,), jnp.inf, dtype=jnp.float32)
        one_v = jnp.ones((16,), dtype=jnp.int32)
        zero_v = jnp.zeros((16,), dtype=jnp.int32)
        k_v = jnp.full((16,), _K, dtype=jnp.int32)
        dt_v = jnp.full((16,), _DT, dtype=jnp.float32)

        def raw_step(row, geom, cnt, tv):
            b_loc, n_off, b_rel_v, n_idx = geom
            v = chunk_v[row, b_loc, pl.ds(n_off, 16)]
            spike = v > 0.0
            plsc.store_scatter(out_v, [b_rel_v, cnt, n_idx], tv, mask=spike)
            return cnt + jnp.where(spike, one_v, zero_v)

        def sat_step(row, geom, cnt, tv):
            b_loc, n_off, b_rel_v, n_idx = geom
            v = chunk_v[row, b_loc, pl.ds(n_off, 16)]
            spike = v > 0.0
            plsc.store_scatter(out_v, [b_rel_v, cnt, n_idx], tv, mask=spike)
            cnt = jnp.minimum(cnt + jnp.where(spike, one_v, zero_v), k_v)
            return cnt, tv + dt_v

        def num_live(cnt):
            return plsc.all_reduce_population_count(cnt < _K)[0]

        def group_geom(g):
            b_rel = g // 8
            n_off = (g % 8) * 16
            b_loc = bh + b_rel
            b_rel_v = jnp.broadcast_to(b_rel, (16,)).astype(jnp.int32)
            n_idx = n_off + lane
            return b_loc, n_off, b_rel_v, n_idx

        def pair_body(p, mask):
            g0 = p * 2
            g1 = g0 + 1
            geom0 = group_geom(g0)
            geom1 = group_geom(g1)

            def pre_cond(state):
                t, c0, c1, tv = state
                return (t < _TP) & (num_live(c0) + num_live(c1) > 0)

            def pre_body(state):
                t, c0, c1, tv = state

                @plsc.parallel_loop(t, t + _SB, unroll=_SB,
                                    carry=(c0, c1, tv))
                def scan(row, state):
                    c0, c1, tv = state
                    c0 = raw_step(row, geom0, c0, tv)
                    c1 = raw_step(row, geom1, c1, tv)
                    return c0, c1, tv + dt_v

                c0, c1, tv = scan
                return t + _SB, jnp.minimum(c0, k_v), jnp.minimum(c1, k_v), tv

            _, c0, c1, _ = lax.while_loop(
                pre_cond, pre_body,
                (jnp.int32(0), zero_v, zero_v,
                 jnp.zeros((16,), jnp.float32)))
            cnt_v[g0, :] = c0
            cnt_v[g1, :] = c1
            s0 = (num_live(c0) > 0).astype(jnp.int32)
            s1 = (num_live(c1) > 0).astype(jnp.int32)
            return mask | (s0 << g0) | (s1 << g1)

        mask = lax.fori_loop(0, _NG // 2, pair_body, jnp.int32(0))

        def rare_cond(state):
            t, mask = state
            return (t < _T) & (mask != 0)

        def rare_body(state):
            t, mask = state
            pltpu.sync_copy(
                x_hbm.at[pl.ds(t, _TB), pl.ds(b0, 8), pl.ds(n0, 128)],
                chunk_v.at[pl.ds(0, _TB)])

            def gb(g, mk):
                def live(mk):
                    geom = group_geom(g)
                    tv0 = jnp.broadcast_to(
                        t.astype(jnp.float32) * _DT, (16,))

                    def inner(i, state):
                        cnt, tv = state
                        return sat_step(i, geom, cnt, tv)

                    cnt, _ = lax.fori_loop(0, _TB, inner, (cnt_v[g, :], tv0))
                    cnt_v[g, :] = cnt
                    done = num_live(cnt) == 0
                    return mk & ~jnp.where(done, jnp.int32(1) << g,
                                           jnp.int32(0))

                return lax.cond((mk >> g) & 1 != 0, live, lambda m: m, mk)

            mask = lax.fori_loop(0, _NG, gb, mask)
            return t + _TB, mask

        _, mask = lax.while_loop(rare_cond, rare_body, (jnp.int32(_TP), mask))

        def fill_body(g, mk):
            def live(mk):
                _, _, b_rel_v, n_idx = group_geom(g)
                cnt = cnt_v[g, :]
                for slot in range(_K):
                    m = cnt <= slot
                    slot_v = jnp.broadcast_to(slot, (16,)).astype(jnp.int32)
                    plsc.store_scatter(out_v, [b_rel_v, slot_v, n_idx],
                                       inf_v, mask=m)
                return mk

            return lax.cond((mk >> g) & 1 != 0, live, lambda m: m, mk)

        lax.fori_loop(0, _NG, fill_body, mask)

        pltpu.sync_copy(
            out_v, out_hbm.at[pl.ds(b0 + bh, 4), :, pl.ds(n0, 128)])

    return decode


_decoder = _make_decoder()


def kernel(spike_input):
    out = _decoder(spike_input)
    return jnp.transpose(out[:, :_K, :], (1, 0, 2))
